# baseline (speedup 1.0000x reference)
import functools

import jax
import jax.numpy as jnp
from jax import lax
from jax.experimental import pallas as pl
from jax.experimental.pallas import tpu as pltpu

_STRIDE = 4
_M = 8
_K = 512
_D = 1024
_T = 4096
_TM = 256
_NT = _T // _TM + _M
_NTOT = _NT * _TM


def _codes_body(tile_model_ref, xs_ref, cb_ref, codes_ref):
    m = tile_model_ref[pl.program_id(0)]
    x = xs_ref[...]
    cb = cb_ref[0]
    mm = lax.dot_general(x, cb, (((1,), (1,)), ((), ())),
                         preferred_element_type=jnp.float32)
    x_sq = jnp.sum(x * x, axis=1, keepdims=True)
    c_sq = jnp.sum(cb * cb, axis=1)[None, :]
    dist = x_sq - 2.0 * mm + c_sq
    minv = jnp.min(dist, axis=1, keepdims=True)
    iot = lax.broadcasted_iota(jnp.int32, dist.shape, 1)
    idx = jnp.min(jnp.where(dist == minv, iot, _K), axis=1)
    codes_ref[0, 0, :] = m * _K + idx


def _schedule(model_ids):
    nb = _T // _STRIDE
    mid = model_ids.astype(jnp.int32)
    perm_b = jnp.argsort(mid, stable=True)
    counts_b = jnp.zeros((_M,), jnp.int32).at[mid].add(1)
    tmb = _TM // _STRIDE
    padc_b = ((counts_b + tmb - 1) // tmb) * tmb
    off_b = jnp.concatenate([jnp.zeros((1,), jnp.int32),
                             jnp.cumsum(counts_b)[:-1].astype(jnp.int32)])
    poff_b = jnp.concatenate([jnp.zeros((1,), jnp.int32),
                              jnp.cumsum(padc_b)[:-1].astype(jnp.int32)])
    jb = jnp.arange(_NTOT // _STRIDE, dtype=jnp.int32)
    mj = jnp.searchsorted(poff_b, jb, side="right").astype(jnp.int32) - 1
    r = jb - poff_b[mj]
    valid_b = r < counts_b[mj]
    spos = jnp.clip(off_b[mj] + jnp.minimum(r, counts_b[mj] - 1), 0, nb - 1)
    gblk = jnp.where(valid_b, perm_b[spos].astype(jnp.int32), 0)
    fr = gblk[:, None] * _STRIDE + jnp.arange(_STRIDE, dtype=jnp.int32)[None, :]
    gidx = jnp.where(valid_b[:, None], fr, 0).reshape(_NTOT)
    trash = _T + (jnp.arange(_NTOT, dtype=jnp.int32) % 8)
    dest = jnp.where(valid_b[:, None], fr, trash.reshape(-1, _STRIDE)).reshape(_NTOT)
    tile_model = jnp.clip(mj[::tmb], 0, _M - 1)
    return gidx, dest, tile_model


def kernel(emb, codebooks, model_ids):
    B, T, D = emb.shape
    flat = emb.reshape(T, D)
    gidx, dest, tile_model = _schedule(model_ids)

    xs = jnp.take(flat, gidx, axis=0)

    grid_spec = pltpu.PrefetchScalarGridSpec(
        num_scalar_prefetch=1,
        grid=(_NT,),
        in_specs=[
            pl.BlockSpec((_TM, _D), lambda i, tm: (i, 0)),
            pl.BlockSpec((1, _K, _D), lambda i, tm: (tm[i], 0, 0)),
        ],
        out_specs=pl.BlockSpec((1, 1, _TM), lambda i, tm: (i, 0, 0)),
    )
    codes3 = pl.pallas_call(
        _codes_body,
        grid_spec=grid_spec,
        out_shape=jax.ShapeDtypeStruct((_NT, 1, _TM), jnp.int32),
    )(tile_model, xs, codebooks)
    codes = codes3.reshape(_NTOT)

    cb_flat = codebooks.reshape(_M * _K, _D)
    q = jnp.take(cb_flat, codes, axis=0)
    out_pad = jnp.zeros((_T + 8, _D), emb.dtype).at[dest].set(q)
    return out_pad[:_T].reshape(B, T, D)


# speedup vs baseline: 1.5604x; 1.5604x over previous
import functools

import jax
import jax.numpy as jnp
from jax import lax
from jax.experimental import pallas as pl
from jax.experimental.pallas import tpu as pltpu
from jax.experimental.pallas import tpu_sc as plsc

_STRIDE = 4
_M = 8
_K = 512
_D = 1024
_T = 4096
_TM = 256
_NT = _T // _TM + _M
_NTOT = _NT * _TM

_NC = 2
_NS = 16
_NW = _NC * _NS
_RPW = _NTOT // _NW
_CH = 96
_NCH = _RPW // _CH

_SC_MESH = plsc.VectorSubcoreMesh(core_axis_name="c", subcore_axis_name="s")


def _schedule(model_ids):
    nb = _T // _STRIDE
    mid = model_ids.astype(jnp.int32)
    perm_b = jnp.argsort(mid, stable=True).astype(jnp.int32)
    counts_b = jnp.zeros((_M,), jnp.int32).at[mid].add(1)
    tmb = _TM // _STRIDE
    padc_b = ((counts_b + tmb - 1) // tmb) * tmb
    off_b = jnp.concatenate([jnp.zeros((1,), jnp.int32),
                             jnp.cumsum(counts_b)[:-1].astype(jnp.int32)])
    poff_b = jnp.concatenate([jnp.zeros((1,), jnp.int32),
                              jnp.cumsum(padc_b)[:-1].astype(jnp.int32)])
    m_big = jnp.argmax(counts_b).astype(jnp.int32)
    jb = jnp.arange(_NTOT // _STRIDE, dtype=jnp.int32)
    mj_raw = jnp.searchsorted(poff_b, jb, side="right").astype(jnp.int32) - 1
    r = jb - poff_b[mj_raw]
    mj = jnp.where(counts_b[mj_raw] > 0, mj_raw, m_big)
    spos = off_b[mj] + jnp.clip(jnp.minimum(r, counts_b[mj] - 1), 0, nb - 1)
    gblk = perm_b[spos]
    fr = gblk[:, None] * _STRIDE + jnp.arange(_STRIDE, dtype=jnp.int32)[None, :]
    gidx = fr.reshape(_NTOT)
    tile_model = mj[::tmb]
    return gidx, gblk, tile_model


_NBTOT = _NTOT // _STRIDE
_BPW = _NBTOT // _NW
_BCH = 24
_BNCH = _BPW // _BCH
_BD = _STRIDE * _D


def _sc_gather_body(src_hbm, idx_hbm, out_hbm, idx_v, buf_v, sem):
    wid = lax.axis_index("s") * _NC + lax.axis_index("c")
    pltpu.sync_copy(idx_hbm.at[pl.ds(wid * _BNCH, _BNCH)], idx_v)
    for c in range(_BNCH):
        pltpu.async_copy(src_hbm.at[idx_v.at[c]], buf_v, sem).wait()
        pltpu.sync_copy(buf_v, out_hbm.at[pl.ds((wid * _BNCH + c) * _BCH, _BCH)])


_sc_gather = functools.partial(
    pl.kernel,
    out_type=jax.ShapeDtypeStruct((_NBTOT, _BD), jnp.float32),
    mesh=_SC_MESH,
    scratch_types=[
        pltpu.VMEM((_BNCH, _BCH), jnp.int32),
        pltpu.VMEM((_BCH, _BD), jnp.float32),
        pltpu.SemaphoreType.DMA,
    ],
)(_sc_gather_body)


def _codes_body(tile_model_ref, xs_ref, cb_ref, codes_ref):
    m = tile_model_ref[pl.program_id(0)]
    x = xs_ref[...]
    cb = cb_ref[0]
    mm = lax.dot_general(x, cb, (((1,), (1,)), ((), ())),
                         preferred_element_type=jnp.float32)
    x_sq = jnp.sum(x * x, axis=1, keepdims=True)
    c_sq = jnp.sum(cb * cb, axis=1)[None, :]
    dist = x_sq - 2.0 * mm + c_sq
    minv = jnp.min(dist, axis=1, keepdims=True)
    iot = lax.broadcasted_iota(jnp.int32, dist.shape, 1)
    idx = jnp.min(jnp.where(dist == minv, iot, _K), axis=1)
    codes_ref[0, 0, :] = m * _K + idx


def _tc_codes(tile_model, xs, codebooks):
    grid_spec = pltpu.PrefetchScalarGridSpec(
        num_scalar_prefetch=1,
        grid=(_NT,),
        in_specs=[
            pl.BlockSpec((_TM, _D), lambda i, tm: (i, 0)),
            pl.BlockSpec((1, _K, _D), lambda i, tm: (tm[i], 0, 0)),
        ],
        out_specs=pl.BlockSpec((1, 1, _TM), lambda i, tm: (i, 0, 0)),
    )
    return pl.pallas_call(
        _codes_body,
        grid_spec=grid_spec,
        out_shape=jax.ShapeDtypeStruct((_NT, 1, _TM), jnp.int32),
    )(tile_model, xs, codebooks)


def _sc_lookup_body(cb_hbm, codes_hbm, dest_hbm, out_hbm,
                    codes_v, dest_v, buf_v, sem):
    wid = lax.axis_index("s") * _NC + lax.axis_index("c")
    pltpu.sync_copy(codes_hbm.at[pl.ds(wid * _NCH, _NCH)], codes_v)
    pltpu.sync_copy(dest_hbm.at[pl.ds(wid * _NCH, _NCH)], dest_v)
    for c in range(_NCH):
        pltpu.async_copy(cb_hbm.at[codes_v.at[c]], buf_v, sem).wait()
        pltpu.async_copy(buf_v, out_hbm.at[dest_v.at[c]], sem).wait()


_sc_lookup = functools.partial(
    pl.kernel,
    out_type=jax.ShapeDtypeStruct((_T, _D), jnp.float32),
    mesh=_SC_MESH,
    scratch_types=[
        pltpu.VMEM((_NCH, _CH), jnp.int32),
        pltpu.VMEM((_NCH, _CH), jnp.int32),
        pltpu.VMEM((_CH, _D), jnp.float32),
        pltpu.SemaphoreType.DMA,
    ],
)(_sc_lookup_body)


def kernel(emb, codebooks, model_ids):
    B, T, D = emb.shape
    flat = emb.reshape(T, D)
    gidx, gblk, tile_model = _schedule(model_ids)
    gidx2 = gidx.reshape(_NW * _NCH, _CH)
    gblk2 = gblk.reshape(_NW * _BNCH, _BCH)

    blocks = flat.reshape(_T // _STRIDE, _BD)
    xs = _sc_gather(blocks, gblk2).reshape(_NTOT, _D)
    codes3 = _tc_codes(tile_model, xs, codebooks)
    codes2 = codes3.reshape(_NW * _NCH, _CH)

    cb_flat = codebooks.reshape(_M * _K, _D)
    out = _sc_lookup(cb_flat, codes2, gidx2)
    return out.reshape(B, T, D)


# speedup vs baseline: 1.6146x; 1.0347x over previous
import functools

import jax
import jax.numpy as jnp
from jax import lax
from jax.experimental import pallas as pl
from jax.experimental.pallas import tpu as pltpu
from jax.experimental.pallas import tpu_sc as plsc

_STRIDE = 4
_M = 8
_K = 512
_D = 1024
_T = 4096
_TM = 256
_NT = _T // _TM + _M
_NTOT = _NT * _TM

_NC = 2
_NS = 16
_NW = _NC * _NS
_RPW = _NTOT // _NW
_CH = 48
_NCH = _RPW // _CH

_SC_MESH = plsc.VectorSubcoreMesh(core_axis_name="c", subcore_axis_name="s")


def _schedule(model_ids):
    nb = _T // _STRIDE
    mid = model_ids.astype(jnp.int32)
    perm_b = jnp.argsort(mid, stable=True).astype(jnp.int32)
    counts_b = jnp.zeros((_M,), jnp.int32).at[mid].add(1)
    tmb = _TM // _STRIDE
    padc_b = ((counts_b + tmb - 1) // tmb) * tmb
    off_b = jnp.concatenate([jnp.zeros((1,), jnp.int32),
                             jnp.cumsum(counts_b)[:-1].astype(jnp.int32)])
    poff_b = jnp.concatenate([jnp.zeros((1,), jnp.int32),
                              jnp.cumsum(padc_b)[:-1].astype(jnp.int32)])
    m_big = jnp.argmax(counts_b).astype(jnp.int32)
    jb = jnp.arange(_NTOT // _STRIDE, dtype=jnp.int32)
    mj_raw = jnp.searchsorted(poff_b, jb, side="right").astype(jnp.int32) - 1
    r = jb - poff_b[mj_raw]
    mj = jnp.where(counts_b[mj_raw] > 0, mj_raw, m_big)
    spos = off_b[mj] + jnp.clip(jnp.minimum(r, counts_b[mj] - 1), 0, nb - 1)
    gblk = perm_b[spos]
    fr = gblk[:, None] * _STRIDE + jnp.arange(_STRIDE, dtype=jnp.int32)[None, :]
    gidx = fr.reshape(_NTOT)
    tile_model = mj[::tmb]
    return gidx, gblk, tile_model


_NBTOT = _NTOT // _STRIDE
_BPW = _NBTOT // _NW
_BCH = 8
_BNCH = _BPW // _BCH
_BD = _STRIDE * _D


def _sc_gather_body(src_hbm, idx_hbm, out_hbm, idx_v, buf0, buf1, sem0, sem1):
    wid = lax.axis_index("s") * _NC + lax.axis_index("c")
    pltpu.sync_copy(idx_hbm.at[wid], idx_v)
    bufs = (buf0, buf1)
    sems = (sem0, sem1)
    cps = [None, None]
    cps[0] = pltpu.async_copy(src_hbm.at[idx_v.at[0]], buf0, sem0)
    for c in range(_BNCH):
        p = c % 2
        cps[p].wait()
        if c + 1 < _BNCH:
            cps[1 - p] = pltpu.async_copy(
                src_hbm.at[idx_v.at[c + 1]], bufs[1 - p], sems[1 - p])
        pltpu.sync_copy(bufs[p], out_hbm.at[pl.ds((wid * _BNCH + c) * _BCH, _BCH)])


_sc_gather = functools.partial(
    pl.kernel,
    out_type=jax.ShapeDtypeStruct((_NBTOT, _BD), jnp.float32),
    mesh=_SC_MESH,
    scratch_types=[
        pltpu.VMEM((_BNCH, _BCH), jnp.int32),
        pltpu.VMEM((_BCH, _BD), jnp.float32),
        pltpu.VMEM((_BCH, _BD), jnp.float32),
        pltpu.SemaphoreType.DMA,
        pltpu.SemaphoreType.DMA,
    ],
)(_sc_gather_body)


def _codes_body(tile_model_ref, xs_ref, cb_ref, codes_ref):
    m = tile_model_ref[pl.program_id(0)]
    x = xs_ref[...]
    cb = cb_ref[0]
    mm = lax.dot_general(x, cb, (((1,), (1,)), ((), ())),
                         preferred_element_type=jnp.float32)
    x_sq = jnp.sum(x * x, axis=1, keepdims=True)
    c_sq = jnp.sum(cb * cb, axis=1)[None, :]
    dist = x_sq - 2.0 * mm + c_sq
    minv = jnp.min(dist, axis=1, keepdims=True)
    iot = lax.broadcasted_iota(jnp.int32, dist.shape, 1)
    idx = jnp.min(jnp.where(dist == minv, iot, _K), axis=1)
    codes_ref[0, 0, :] = m * _K + idx


def _tc_codes(tile_model, xs, codebooks):
    grid_spec = pltpu.PrefetchScalarGridSpec(
        num_scalar_prefetch=1,
        grid=(_NT,),
        in_specs=[
            pl.BlockSpec((_TM, _D), lambda i, tm: (i, 0)),
            pl.BlockSpec((1, _K, _D), lambda i, tm: (tm[i], 0, 0)),
        ],
        out_specs=pl.BlockSpec((1, 1, _TM), lambda i, tm: (i, 0, 0)),
    )
    return pl.pallas_call(
        _codes_body,
        grid_spec=grid_spec,
        out_shape=jax.ShapeDtypeStruct((_NT, 1, _TM), jnp.int32),
    )(tile_model, xs, codebooks)


def _sc_lookup_body(cb_hbm, codes_hbm, dest_hbm, out_hbm,
                    codes_v, dest_v, buf0, buf1, sem0, sem1):
    wid = lax.axis_index("s") * _NC + lax.axis_index("c")
    pltpu.sync_copy(codes_hbm.at[wid], codes_v)
    pltpu.sync_copy(dest_hbm.at[wid], dest_v)
    bufs = (buf0, buf1)
    sems = (sem0, sem1)
    cps = [None, None]
    cps[0] = pltpu.async_copy(cb_hbm.at[codes_v.at[0]], buf0, sem0)
    for c in range(_NCH):
        p = c % 2
        cps[p].wait()
        if c + 1 < _NCH:
            cps[1 - p] = pltpu.async_copy(
                cb_hbm.at[codes_v.at[c + 1]], bufs[1 - p], sems[1 - p])
        pltpu.async_copy(bufs[p], out_hbm.at[dest_v.at[c]], sems[p]).wait()


_sc_lookup = functools.partial(
    pl.kernel,
    out_type=jax.ShapeDtypeStruct((_T, _D), jnp.float32),
    mesh=_SC_MESH,
    scratch_types=[
        pltpu.VMEM((_NCH, _CH), jnp.int32),
        pltpu.VMEM((_NCH, _CH), jnp.int32),
        pltpu.VMEM((_CH, _D), jnp.float32),
        pltpu.VMEM((_CH, _D), jnp.float32),
        pltpu.SemaphoreType.DMA,
        pltpu.SemaphoreType.DMA,
    ],
)(_sc_lookup_body)


def kernel(emb, codebooks, model_ids):
    B, T, D = emb.shape
    flat = emb.reshape(T, D)
    gidx, gblk, tile_model = _schedule(model_ids)
    gidx3 = gidx.reshape(_NW, _NCH, _CH)
    gblk3 = gblk.reshape(_NW, _BNCH, _BCH)

    blocks = flat.reshape(_T // _STRIDE, _BD)
    xs = _sc_gather(blocks, gblk3).reshape(_NTOT, _D)
    codes3 = _tc_codes(tile_model, xs, codebooks)
    codes_w = codes3.reshape(_NW, _NCH, _CH)

    cb_flat = codebooks.reshape(_M * _K, _D)
    out = _sc_lookup(cb_flat, codes_w, gidx3)
    return out.reshape(B, T, D)


# speedup vs baseline: 2.4176x; 1.4974x over previous
import functools

import jax
import jax.numpy as jnp
from jax import lax
from jax.experimental import pallas as pl
from jax.experimental.pallas import tpu as pltpu
from jax.experimental.pallas import tpu_sc as plsc

_STRIDE = 4
_M = 8
_K = 512
_D = 1024
_T = 4096
_TM = 256
_NT = _T // _TM + _M
_NTOT = _NT * _TM
_NBTOT = _NTOT // _STRIDE

_NC = 2
_NS = 16
_NW = _NC * _NS
_RPW = _NTOT // _NW
_CH = 48
_NCH = _RPW // _CH

def _sc_mesh():
    return plsc.VectorSubcoreMesh(core_axis_name="c", subcore_axis_name="s")


def _schedule(model_ids):
    mid = model_ids.astype(jnp.int32)
    nb = _T // _STRIDE
    tmb = _TM // _STRIDE
    oh = (mid[:, None] == jnp.arange(_M, dtype=jnp.int32)[None, :])
    csum = jnp.cumsum(oh.astype(jnp.int32), axis=0)
    counts = csum[-1]
    rank = jnp.sum(csum * oh, axis=1) - 1
    padc = ((counts + tmb - 1) // tmb) * tmb
    poff = jnp.concatenate([jnp.zeros((1,), jnp.int32),
                            jnp.cumsum(padc)[:-1].astype(jnp.int32)])
    slot = poff[mid] + rank
    arrblk = jnp.full((_NBTOT,), -1, jnp.int32).at[slot].set(
        jnp.arange(nb, dtype=jnp.int32))
    iota = jnp.arange(_NBTOT, dtype=jnp.int32)
    posf = lax.cummax(jnp.where(arrblk >= 0, iota, -1), axis=0)
    gblk = arrblk[posf]
    fr = gblk[:, None] * _STRIDE + jnp.arange(_STRIDE, dtype=jnp.int32)[None, :]
    gidx = fr.reshape(_NTOT)
    tile_model = mid[gblk[::tmb]]
    return gidx, tile_model


def _sc_gather_body(src_hbm, idx_hbm, out_hbm, idx_v, buf0, buf1, sem0, sem1):
    wid = lax.axis_index("s") * _NC + lax.axis_index("c")
    pltpu.sync_copy(idx_hbm.at[wid], idx_v)
    bufs = (buf0, buf1)
    sems = (sem0, sem1)
    cps = [None, None]
    cps[0] = pltpu.async_copy(src_hbm.at[idx_v.at[0]], buf0, sem0)
    for c in range(_NCH):
        p = c % 2
        cps[p].wait()
        if c + 1 < _NCH:
            cps[1 - p] = pltpu.async_copy(
                src_hbm.at[idx_v.at[c + 1]], bufs[1 - p], sems[1 - p])
        pltpu.sync_copy(bufs[p], out_hbm.at[pl.ds((wid * _NCH + c) * _CH, _CH)])


@functools.cache
def _sc_gather():
    return pl.kernel(
        _sc_gather_body,
        out_type=jax.ShapeDtypeStruct((_NTOT, _D), jnp.float32),
        mesh=_sc_mesh(),
        scratch_types=[
            pltpu.VMEM((_NCH, _CH), jnp.int32),
            pltpu.VMEM((_CH, _D), jnp.float32),
            pltpu.VMEM((_CH, _D), jnp.float32),
            pltpu.SemaphoreType.DMA,
            pltpu.SemaphoreType.DMA,
        ],
    )


def _codes_body(tile_model_ref, xs_ref, cb_ref, codes_ref):
    m = tile_model_ref[pl.program_id(0)]
    x = xs_ref[...]
    cb = cb_ref[0]
    mm = lax.dot_general(x, cb, (((1,), (1,)), ((), ())),
                         preferred_element_type=jnp.float32)
    x_sq = jnp.sum(x * x, axis=1, keepdims=True)
    c_sq = jnp.sum(cb * cb, axis=1)[None, :]
    dist = x_sq - 2.0 * mm + c_sq
    minv = jnp.min(dist, axis=1, keepdims=True)
    iot = lax.broadcasted_iota(jnp.int32, dist.shape, 1)
    idx = jnp.min(jnp.where(dist == minv, iot, _K), axis=1)
    codes_ref[0, 0, :] = m * _K + idx


def _tc_codes(tile_model, xs, codebooks):
    grid_spec = pltpu.PrefetchScalarGridSpec(
        num_scalar_prefetch=1,
        grid=(_NT,),
        in_specs=[
            pl.BlockSpec((_TM, _D), lambda i, tm: (i, 0)),
            pl.BlockSpec((1, _K, _D), lambda i, tm: (tm[i], 0, 0)),
        ],
        out_specs=pl.BlockSpec((1, 1, _TM), lambda i, tm: (i, 0, 0)),
    )
    return pl.pallas_call(
        _codes_body,
        grid_spec=grid_spec,
        out_shape=jax.ShapeDtypeStruct((_NT, 1, _TM), jnp.int32),
    )(tile_model, xs, codebooks)


def _sc_lookup_body(cb_hbm, codes_hbm, dest_hbm, out_hbm,
                    codes_v, dest_v, buf0, buf1, sem0, sem1):
    wid = lax.axis_index("s") * _NC + lax.axis_index("c")
    pltpu.sync_copy(codes_hbm.at[wid], codes_v)
    pltpu.sync_copy(dest_hbm.at[wid], dest_v)
    bufs = (buf0, buf1)
    sems = (sem0, sem1)
    cps = [None, None]
    cps[0] = pltpu.async_copy(cb_hbm.at[codes_v.at[0]], buf0, sem0)
    for c in range(_NCH):
        p = c % 2
        cps[p].wait()
        if c + 1 < _NCH:
            cps[1 - p] = pltpu.async_copy(
                cb_hbm.at[codes_v.at[c + 1]], bufs[1 - p], sems[1 - p])
        pltpu.async_copy(bufs[p], out_hbm.at[dest_v.at[c]], sems[p]).wait()


@functools.cache
def _sc_lookup():
    return pl.kernel(
        _sc_lookup_body,
        out_type=jax.ShapeDtypeStruct((_T, _D), jnp.float32),
        mesh=_sc_mesh(),
        scratch_types=[
            pltpu.VMEM((_NCH, _CH), jnp.int32),
            pltpu.VMEM((_NCH, _CH), jnp.int32),
            pltpu.VMEM((_CH, _D), jnp.float32),
            pltpu.VMEM((_CH, _D), jnp.float32),
            pltpu.SemaphoreType.DMA,
            pltpu.SemaphoreType.DMA,
        ],
    )


def kernel(emb, codebooks, model_ids):
    B, T, D = emb.shape
    flat = emb.reshape(T, D)
    gidx, tile_model = _schedule(model_ids)
    gidx3 = gidx.reshape(_NW, _NCH, _CH)

    xs = _sc_gather()(flat, gidx3)
    codes3 = _tc_codes(tile_model, xs, codebooks)
    codes_w = codes3.reshape(_NW, _NCH, _CH)

    cb_flat = codebooks.reshape(_M * _K, _D)
    out = _sc_lookup()(cb_flat, codes_w, gidx3)
    return out.reshape(B, T, D)


# speedup vs baseline: 2.5034x; 1.0355x over previous
import functools

import jax
import jax.numpy as jnp
from jax import lax
from jax.experimental import pallas as pl
from jax.experimental.pallas import tpu as pltpu
from jax.experimental.pallas import tpu_sc as plsc

_STRIDE = 4
_M = 8
_K = 512
_D = 1024
_T = 4096
_TM = 256
_NT = _T // _TM + _M
_NTOT = _NT * _TM
_NBTOT = _NTOT // _STRIDE

_NC = 2
_NS = 16
_NW = _NC * _NS
_CH = 48
_NCH = _NTOT // _NW // _CH
_G = 2
_NTG = _NT // _G
_GTOT = _NTOT // _G
_GNCH = _GTOT // _NW // _CH

def _sc_mesh():
    return plsc.VectorSubcoreMesh(core_axis_name="c", subcore_axis_name="s")


def _schedule(model_ids):
    mid = model_ids.astype(jnp.int32)
    nb = _T // _STRIDE
    tmb = _TM // _STRIDE
    oh = (mid[:, None] == jnp.arange(_M, dtype=jnp.int32)[None, :])
    csum = jnp.cumsum(oh.astype(jnp.int32), axis=0)
    counts = csum[-1]
    rank = jnp.sum(csum * oh, axis=1) - 1
    padc = ((counts + tmb - 1) // tmb) * tmb
    poff = jnp.concatenate([jnp.zeros((1,), jnp.int32),
                            jnp.cumsum(padc)[:-1].astype(jnp.int32)])
    slot = poff[mid] + rank
    arrblk = jnp.full((_NBTOT,), -1, jnp.int32).at[slot].set(
        jnp.arange(nb, dtype=jnp.int32))
    iota = jnp.arange(_NBTOT, dtype=jnp.int32)
    posf = lax.cummax(jnp.where(arrblk >= 0, iota, -1), axis=0)
    gblk = arrblk[posf]
    fr = gblk[:, None] * _STRIDE + jnp.arange(_STRIDE, dtype=jnp.int32)[None, :]
    gidx = fr.reshape(_NTOT)
    tile_model = mid[gblk[::tmb]]
    return gidx, tile_model


def _sc_gather_body(nch, src_hbm, idx_hbm, out_hbm, idx_v, buf0, buf1,
                    sem0, sem1):
    wid = lax.axis_index("s") * _NC + lax.axis_index("c")
    pltpu.sync_copy(idx_hbm.at[wid], idx_v)
    bufs = (buf0, buf1)
    sems = (sem0, sem1)
    cps = [None, None]
    cps[0] = pltpu.async_copy(src_hbm.at[idx_v.at[0]], buf0, sem0)
    for c in range(nch):
        p = c % 2
        cps[p].wait()
        if c + 1 < nch:
            cps[1 - p] = pltpu.async_copy(
                src_hbm.at[idx_v.at[c + 1]], bufs[1 - p], sems[1 - p])
        pltpu.sync_copy(bufs[p], out_hbm.at[pl.ds((wid * nch + c) * _CH, _CH)])


@functools.cache
def _sc_gather(ntot, nch):
    return pl.kernel(
        functools.partial(_sc_gather_body, nch),
        out_type=jax.ShapeDtypeStruct((ntot, _D), jnp.float32),
        mesh=_sc_mesh(),
        scratch_types=[
            pltpu.VMEM((nch, _CH), jnp.int32),
            pltpu.VMEM((_CH, _D), jnp.float32),
            pltpu.VMEM((_CH, _D), jnp.float32),
            pltpu.SemaphoreType.DMA,
            pltpu.SemaphoreType.DMA,
        ],
    )


def _codes_body(tile_model_ref, xs_ref, cb_ref, codes_ref):
    m = tile_model_ref[pl.program_id(0)]
    x = xs_ref[...]
    cb = cb_ref[0]
    mm = lax.dot_general(x, cb, (((1,), (1,)), ((), ())),
                         preferred_element_type=jnp.float32)
    x_sq = jnp.sum(x * x, axis=1, keepdims=True)
    c_sq = jnp.sum(cb * cb, axis=1)[None, :]
    dist = x_sq - 2.0 * mm + c_sq
    minv = jnp.min(dist, axis=1, keepdims=True)
    iot = lax.broadcasted_iota(jnp.int32, dist.shape, 1)
    idx = jnp.min(jnp.where(dist == minv, iot, _K), axis=1)
    codes_ref[0, 0, :] = m * _K + idx


def _tc_codes(nt, tile_model, xs, codebooks):
    grid_spec = pltpu.PrefetchScalarGridSpec(
        num_scalar_prefetch=1,
        grid=(nt,),
        in_specs=[
            pl.BlockSpec((_TM, _D), lambda i, tm: (i, 0)),
            pl.BlockSpec((1, _K, _D), lambda i, tm: (tm[i], 0, 0)),
        ],
        out_specs=pl.BlockSpec((1, 1, _TM), lambda i, tm: (i, 0, 0)),
    )
    return pl.pallas_call(
        _codes_body,
        grid_spec=grid_spec,
        out_shape=jax.ShapeDtypeStruct((nt, 1, _TM), jnp.int32),
    )(tile_model, xs, codebooks)


def _sc_lookup_body(cb_hbm, codes_hbm, dest_hbm, out_hbm,
                    codes_v, dest_v, buf0, buf1, sem0, sem1):
    wid = lax.axis_index("s") * _NC + lax.axis_index("c")
    pltpu.sync_copy(codes_hbm.at[wid], codes_v)
    pltpu.sync_copy(dest_hbm.at[wid], dest_v)
    bufs = (buf0, buf1)
    sems = (sem0, sem1)
    cps = [None, None]
    cps[0] = pltpu.async_copy(cb_hbm.at[codes_v.at[0]], buf0, sem0)
    for c in range(_NCH):
        p = c % 2
        cps[p].wait()
        if c + 1 < _NCH:
            cps[1 - p] = pltpu.async_copy(
                cb_hbm.at[codes_v.at[c + 1]], bufs[1 - p], sems[1 - p])
        pltpu.async_copy(bufs[p], out_hbm.at[dest_v.at[c]], sems[p]).wait()


@functools.cache
def _sc_lookup():
    return pl.kernel(
        _sc_lookup_body,
        out_type=jax.ShapeDtypeStruct((_T, _D), jnp.float32),
        mesh=_sc_mesh(),
        scratch_types=[
            pltpu.VMEM((_NCH, _CH), jnp.int32),
            pltpu.VMEM((_NCH, _CH), jnp.int32),
            pltpu.VMEM((_CH, _D), jnp.float32),
            pltpu.VMEM((_CH, _D), jnp.float32),
            pltpu.SemaphoreType.DMA,
            pltpu.SemaphoreType.DMA,
        ],
    )


def kernel(emb, codebooks, model_ids):
    B, T, D = emb.shape
    flat = emb.reshape(T, D)
    gidx, tile_model = _schedule(model_ids)
    gidx_g = gidx.reshape(_G, _NW, _GNCH, _CH)

    codes = []
    for h in range(_G):
        xs_h = _sc_gather(_GTOT, _GNCH)(flat, gidx_g[h])
        tm_h = lax.dynamic_slice_in_dim(tile_model, h * _NTG, _NTG)
        codes.append(_tc_codes(_NTG, tm_h, xs_h, codebooks))
    codes_w = jnp.stack(codes).reshape(_NW, _NCH, _CH)

    cb_flat = codebooks.reshape(_M * _K, _D)
    out = _sc_lookup()(cb_flat, codes_w, gidx.reshape(_NW, _NCH, _CH))
    return out.reshape(B, T, D)


# speedup vs baseline: 2.8475x; 1.1375x over previous
import functools

import jax
import jax.numpy as jnp
from jax import lax
from jax.experimental import pallas as pl
from jax.experimental.pallas import tpu as pltpu
from jax.experimental.pallas import tpu_sc as plsc

_STRIDE = 4
_M = 8
_K = 512
_D = 1024
_T = 4096
_TM = 256
_NT = _T // _TM + _M
_NTOT = _NT * _TM
_NBTOT = _NTOT // _STRIDE

_NC = 2
_NS = 16
_NW = _NC * _NS
_RPW = _NTOT // _NW
_CH = 48
_NCH = _RPW // _CH

def _sc_mesh():
    return plsc.VectorSubcoreMesh(core_axis_name="c", subcore_axis_name="s")


def _schedule(model_ids):
    mid = model_ids.astype(jnp.int32)
    nb = _T // _STRIDE
    tmb = _TM // _STRIDE
    oh = (mid[:, None] == jnp.arange(_M, dtype=jnp.int32)[None, :])
    csum = jnp.cumsum(oh.astype(jnp.int32), axis=0)
    counts = csum[-1]
    rank = jnp.sum(csum * oh, axis=1) - 1
    padc = ((counts + tmb - 1) // tmb) * tmb
    poff = jnp.concatenate([jnp.zeros((1,), jnp.int32),
                            jnp.cumsum(padc)[:-1].astype(jnp.int32)])
    slot = poff[mid] + rank
    arrblk = jnp.full((_NBTOT,), -1, jnp.int32).at[slot].set(
        jnp.arange(nb, dtype=jnp.int32))
    iota = jnp.arange(_NBTOT, dtype=jnp.int32)
    posf = lax.cummax(jnp.where(arrblk >= 0, iota, -1), axis=0)
    gblk = arrblk[posf]
    fr = gblk[:, None] * _STRIDE + jnp.arange(_STRIDE, dtype=jnp.int32)[None, :]
    gidx = fr.reshape(_NTOT)
    tile_model = mid[gblk[::tmb]]
    return gidx, gblk, tile_model


_TMB = _TM // _STRIDE


def _issue_tile(gblk_ref, emb_ref, xbuf, sem, i):
    for b in range(_TMB):
        blk = gblk_ref[i * _TMB + b]
        pltpu.make_async_copy(
            emb_ref.at[pl.ds(blk * _STRIDE, _STRIDE)],
            xbuf.at[pl.ds(b * _STRIDE, _STRIDE)],
            sem,
        ).start()


def _wait_tile(gblk_ref, emb_ref, xbuf, sem, i):
    for b in range(_TMB):
        blk = gblk_ref[i * _TMB + b]
        pltpu.make_async_copy(
            emb_ref.at[pl.ds(blk * _STRIDE, _STRIDE)],
            xbuf.at[pl.ds(b * _STRIDE, _STRIDE)],
            sem,
        ).wait()


def _codes_body(tile_model_ref, gblk_ref, emb_ref, cb_ref, codes_ref,
                xb0, xb1, sem0, sem1):
    i = pl.program_id(0)
    bufs = (xb0, xb1)
    sems = (sem0, sem1)

    @pl.when(i == 0)
    def _prologue():
        _issue_tile(gblk_ref, emb_ref, xb0, sem0, 0)

    @pl.when((i + 1 < _NT) & (i % 2 == 0))
    def _next_even():
        _issue_tile(gblk_ref, emb_ref, xb1, sem1, i + 1)

    @pl.when((i + 1 < _NT) & (i % 2 == 1))
    def _next_odd():
        _issue_tile(gblk_ref, emb_ref, xb0, sem0, i + 1)

    m = tile_model_ref[i]
    cb = cb_ref[0]
    c_sq = jnp.sum(cb * cb, axis=1)[None, :]

    def _tile(xbuf, sem, par):
        _wait_tile(gblk_ref, emb_ref, xbuf, sem, i)
        x = xbuf[...]
        mm = lax.dot_general(x, cb, (((1,), (1,)), ((), ())),
                             preferred_element_type=jnp.float32)
        x_sq = jnp.sum(x * x, axis=1, keepdims=True)
        dist = x_sq - 2.0 * mm + c_sq
        minv = jnp.min(dist, axis=1, keepdims=True)
        iot = lax.broadcasted_iota(jnp.int32, dist.shape, 1)
        idx = jnp.min(jnp.where(dist == minv, iot, _K), axis=1)
        codes_ref[0, 0, :] = m * _K + idx

    @pl.when(i % 2 == 0)
    def _even():
        _tile(xb0, sem0, 0)

    @pl.when(i % 2 == 1)
    def _odd():
        _tile(xb1, sem1, 1)


def _tc_codes(tile_model, gblk, emb_flat, codebooks):
    grid_spec = pltpu.PrefetchScalarGridSpec(
        num_scalar_prefetch=2,
        grid=(_NT,),
        in_specs=[
            pl.BlockSpec(memory_space=pl.ANY),
            pl.BlockSpec((1, _K, _D), lambda i, tm, gb: (tm[i], 0, 0)),
        ],
        out_specs=pl.BlockSpec((1, 1, _TM), lambda i, tm, gb: (i, 0, 0)),
        scratch_shapes=[
            pltpu.VMEM((_TM, _D), jnp.float32),
            pltpu.VMEM((_TM, _D), jnp.float32),
            pltpu.SemaphoreType.DMA,
            pltpu.SemaphoreType.DMA,
        ],
    )
    return pl.pallas_call(
        _codes_body,
        grid_spec=grid_spec,
        out_shape=jax.ShapeDtypeStruct((_NT, 1, _TM), jnp.int32),
    )(tile_model, gblk, emb_flat, codebooks)


def _sc_lookup_body(cb_hbm, codes_hbm, dest_hbm, out_hbm,
                    codes_v, dest_v, buf0, buf1, sem0, sem1):
    wid = lax.axis_index("s") * _NC + lax.axis_index("c")
    pltpu.sync_copy(codes_hbm.at[wid], codes_v)
    pltpu.sync_copy(dest_hbm.at[wid], dest_v)
    bufs = (buf0, buf1)
    sems = (sem0, sem1)
    cps = [None, None]
    cps[0] = pltpu.async_copy(cb_hbm.at[codes_v.at[0]], buf0, sem0)
    for c in range(_NCH):
        p = c % 2
        cps[p].wait()
        if c + 1 < _NCH:
            cps[1 - p] = pltpu.async_copy(
                cb_hbm.at[codes_v.at[c + 1]], bufs[1 - p], sems[1 - p])
        pltpu.async_copy(bufs[p], out_hbm.at[dest_v.at[c]], sems[p]).wait()


@functools.cache
def _sc_lookup():
    return pl.kernel(
        _sc_lookup_body,
        out_type=jax.ShapeDtypeStruct((_T, _D), jnp.float32),
        mesh=_sc_mesh(),
        scratch_types=[
            pltpu.VMEM((_NCH, _CH), jnp.int32),
            pltpu.VMEM((_NCH, _CH), jnp.int32),
            pltpu.VMEM((_CH, _D), jnp.float32),
            pltpu.VMEM((_CH, _D), jnp.float32),
            pltpu.SemaphoreType.DMA,
            pltpu.SemaphoreType.DMA,
        ],
    )


def kernel(emb, codebooks, model_ids):
    B, T, D = emb.shape
    flat = emb.reshape(T, D)
    gidx, gblk, tile_model = _schedule(model_ids)

    codes3 = _tc_codes(tile_model, gblk, flat, codebooks)
    codes_w = codes3.reshape(_NW, _NCH, _CH)

    cb_flat = codebooks.reshape(_M * _K, _D)
    out = _sc_lookup()(cb_flat, codes_w, gidx.reshape(_NW, _NCH, _CH))
    return out.reshape(B, T, D)
